# Initial kernel scaffold; baseline (speedup 1.0000x reference)
import jax
import jax.numpy as jnp
from jax.experimental import pallas as pl


def kernel(x, attributes, residues, weight, bias, pixel_to_node, ancestors):
    raise NotImplementedError("write your pallas kernel here")


# speedup vs baseline: 117.1837x; 117.1837x over previous
import functools
import jax
import jax.numpy as jnp
from jax import lax
from jax.experimental import pallas as pl
from jax.experimental.pallas import tpu as pltpu, tpu_sc as plsc

N_NODES = 100000
DEPTH = 16
L = 16
NC, NS = 2, 16
NW = NC * NS

NPAD = 100352
K1 = 784
N_K1 = NPAD // (NS * K1)
K2 = 784
N_K2 = NPAD // (NW * K2)

HW = 512 * 512
PIX_PER_TILE = HW // NW

_MESH = plsc.VectorSubcoreMesh(
    core_axis_name="c", subcore_axis_name="s", num_cores=NC, num_subcores=NS
)
_PARAMS = pltpu.CompilerParams(
    needs_layout_passes=False, use_tc_tiling_on_sc=False)


def _node_val_body(attr_c, res_p, wsb, anc_c, nv_out,
                   contrib, attr_sub, res_sub, wsb_v, anc_blk, nodeval, spmem):
    cid = lax.axis_index("c")
    sid = lax.axis_index("s")
    wid = sid * NC + cid

    pltpu.sync_copy(wsb, wsb_v)
    w0 = wsb_v[0]
    w1 = wsb_v[1]
    w2 = wsb_v[2]
    bv = wsb_v[3]
    for k in range(N_K1):
        ch = sid * N_K1 + k
        pltpu.sync_copy(attr_c.at[ch], attr_sub)
        pltpu.sync_copy(res_p.at[pl.ds(ch * K1, K1)], res_sub)
        base = ch * K1

        def p1_body(g, _, base=base):
            off = g * L
            a0 = attr_sub[0, pl.ds(off, L)]
            a1 = attr_sub[1, pl.ds(off, L)]
            a2 = attr_sub[2, pl.ds(off, L)]
            r = res_sub[pl.ds(off, L)]
            logit = a0 * w0 + a1 * w1 + a2 * w2 + bv
            score = 1.0 / (1.0 + jnp.exp(-logit))
            contrib[pl.ds(base + off, L)] = score * r
            return 0

        lax.fori_loop(0, K1 // L, p1_body, 0)

    own = pl.ds(sid * (NPAD // NS), NPAD // NS)
    pltpu.sync_copy(contrib.at[own], spmem.at[own])
    plsc.subcore_barrier()
    pltpu.sync_copy(spmem, contrib)

    for s in range(N_K2):
        ch2 = wid * N_K2 + s
        pltpu.sync_copy(anc_c.at[ch2], anc_blk)

        def p2_body(g, _):
            off = g * L
            acc = plsc.load_gather(contrib, [anc_blk[0, pl.ds(off, L)]])
            for d in range(1, DEPTH):
                acc = acc + plsc.load_gather(contrib, [anc_blk[d, pl.ds(off, L)]])
            nodeval[pl.ds(off, L)] = acc
            return 0

        lax.fori_loop(0, K2 // L, p2_body, 0)
        pltpu.sync_copy(nodeval, nv_out.at[pl.ds(ch2 * K2, K2)])


def _pixel_body(nv, p2n, out_flat, nv_full, p2n_blk, y_blk):
    cid = lax.axis_index("c")
    sid = lax.axis_index("s")
    wid = sid * NC + cid

    pltpu.sync_copy(nv, nv_full)
    pltpu.sync_copy(p2n.at[pl.ds(wid * PIX_PER_TILE, PIX_PER_TILE)], p2n_blk)

    def body(g, _):
        off = g * L
        y_blk[pl.ds(off, L)] = plsc.load_gather(
            nv_full, [p2n_blk[pl.ds(off, L)]])
        return 0

    lax.fori_loop(0, PIX_PER_TILE // L, body, 0)

    for bc in range(6):
        pltpu.sync_copy(
            y_blk, out_flat.at[pl.ds(bc * HW + wid * PIX_PER_TILE, PIX_PER_TILE)])


_node_val_kernel = functools.partial(
    pl.kernel,
    out_type=jax.ShapeDtypeStruct((NPAD,), jnp.float32),
    mesh=_MESH,
    scratch_types=[
        pltpu.VMEM((NPAD,), jnp.float32),
        pltpu.VMEM((3, K1), jnp.float32),
        pltpu.VMEM((K1,), jnp.float32),
        pltpu.VMEM((4, L), jnp.float32),
        pltpu.VMEM((DEPTH, K2), jnp.int32),
        pltpu.VMEM((K2,), jnp.float32),
        pltpu.VMEM_SHARED((NPAD,), jnp.float32),
    ],
    compiler_params=_PARAMS,
)(_node_val_body)

_pixel_kernel = functools.partial(
    pl.kernel,
    out_type=jax.ShapeDtypeStruct((6 * HW,), jnp.float32),
    mesh=_MESH,
    scratch_types=[
        pltpu.VMEM((NPAD,), jnp.float32),
        pltpu.VMEM((PIX_PER_TILE,), jnp.int32),
        pltpu.VMEM((PIX_PER_TILE,), jnp.float32),
    ],
    compiler_params=_PARAMS,
)(_pixel_body)


@jax.jit
def kernel(x, attributes, residues, weight, bias, pixel_to_node, ancestors):
    Bn, Cn, Hn, Wn = x.shape
    pad = NPAD - N_NODES

    attr_p = jnp.pad(attributes, ((0, pad), (0, 0)))
    attr_c = attr_p.T.reshape(3, NPAD // K1, K1).transpose(1, 0, 2)
    res_p = jnp.pad(residues, (0, pad))
    wsb = jnp.concatenate(
        [jnp.broadcast_to(weight[:, None], (3, L)),
         jnp.broadcast_to(bias[:, None], (1, L))], axis=0)
    anc_p = jnp.pad(ancestors.astype(jnp.int32), ((0, pad), (0, 0)))
    anc_c = anc_p.reshape(NPAD // K2, K2, DEPTH).transpose(0, 2, 1)
    p2n = pixel_to_node.astype(jnp.int32)

    node_val = _node_val_kernel(attr_c, res_p, wsb, anc_c)
    out_flat = _pixel_kernel(node_val, p2n)
    return out_flat.reshape(Bn, Cn, Hn, Wn)
